# baseline (speedup 1.0000x reference)
import functools

import jax
import jax.numpy as jnp
from jax import lax
from jax.experimental import pallas as pl
from jax.experimental.pallas import tpu as pltpu
from jax.experimental.pallas import tpu_sc as plsc

N = 100000
IN_FEATS = 128
NCAT = 26
V = 100000
D = 16
OUT = 128
EMB_COLS = NCAT * D
GPC = V // 16

NW = 32
BPW = 3328
NPAD = NW * BPW
R = 128
NCHUNK = BPW // R
BN = 2048


def _sc_gather(tables8, idx_t):
    mesh = plsc.VectorSubcoreMesh(core_axis_name="c", subcore_axis_name="s")
    nc = 2

    @functools.partial(
        pl.kernel,
        mesh=mesh,
        out_type=jax.ShapeDtypeStruct((EMB_COLS, NPAD), jnp.float32),
        scratch_types=[
            pltpu.VMEM((2, R), jnp.int32),
            pltpu.VMEM((2, R), jnp.int32),
            pltpu.VMEM((2, R, 128), jnp.int32),
            pltpu.VMEM((2, D, R), jnp.float32),
            pltpu.VMEM_SHARED((GPC, 128), jnp.int32),
            pltpu.SemaphoreType.DMA,
            pltpu.SemaphoreType.DMA,
        ],
        compiler_params=pltpu.CompilerParams(needs_layout_passes=False),
    )
    def k(t8_hbm, idx_hbm, emb_hbm, idx_v, gidx_v, sup_v, pk_v, tab_sh,
          gsem, wsem):
        cid = lax.axis_index("c")
        sid = lax.axis_index("s")
        wid = sid * nc + cid
        base = pl.multiple_of(wid * BPW, 128)
        iota16 = lax.iota(jnp.int32, 16)

        def load_prep_fire(c, r):
            h = r & 1
            off = pl.multiple_of(base + r * R, 128)
            pltpu.sync_copy(idx_hbm.at[c].at[pl.ds(off, R)], idx_v.at[h])

            def prep8(i8, _):
                v = idx_v[h, pl.ds(i8 * 16, 16)]
                gidx_v[h, pl.ds(i8 * 16, 16)] = v >> 4
                return _

            lax.fori_loop(0, R // 16, prep8, 0)
            pltpu.async_copy(tab_sh.at[gidx_v.at[h]], sup_v.at[h], gsem)

        def drain_gather():
            pltpu.make_async_copy(
                t8_hbm.at[0].at[pl.ds(0, R), :], sup_v.at[0], gsem
            ).wait()

        def drain_wb():
            pltpu.make_async_copy(
                t8_hbm.at[0].at[pl.ds(0, D), :], pk_v.at[0], wsem
            ).wait()

        def compact_wb(c, r):
            h = r & 1

            def body(i8, _):
                rows = i8 * 16 + iota16
                kv = (idx_v[h, pl.ds(i8 * 16, 16)] & 15) * 8
                for j in range(D // 2):
                    vals = plsc.load_gather(sup_v.at[h], [rows, kv + j])
                    lo = plsc.bitcast(vals << 16, jnp.float32)
                    hi = plsc.bitcast(vals & jnp.int32(-65536), jnp.float32)
                    plsc.store_scatter(
                        pk_v.at[h],
                        [jnp.broadcast_to(jnp.int32(2 * j), (16,)), rows],
                        lo,
                    )
                    plsc.store_scatter(
                        pk_v.at[h],
                        [jnp.broadcast_to(jnp.int32(2 * j + 1), (16,)), rows],
                        hi,
                    )
                return _

            lax.fori_loop(0, R // 16, body, 0)
            off = pl.multiple_of(base + r * R, 128)
            pltpu.async_copy(
                pk_v.at[h],
                emb_hbm.at[pl.ds(pl.multiple_of(c * D, 8), D), pl.ds(off, R)],
                wsem,
            )

        def col(c, carry):
            @pl.when(sid == 0)
            def _():
                pltpu.sync_copy(t8_hbm.at[c], tab_sh)

            plsc.subcore_barrier()

            load_prep_fire(c, 0)

            def step(r, cc):
                @pl.when(r + 1 < NCHUNK)
                def _():
                    load_prep_fire(c, r + 1)

                drain_gather()

                @pl.when(r >= 2)
                def _():
                    drain_wb()

                compact_wb(c, r)
                return cc

            lax.fori_loop(0, NCHUNK, step, 0)
            drain_wb()
            drain_wb()
            plsc.subcore_barrier()
            return carry

        lax.fori_loop(0, NCAT, col, 0)

    return k(tables8, idx_t)


def _mm_body(feat_ref, embt_ref, wf_ref, w2_ref, b_ref, out_ref):
    acc = jnp.dot(feat_ref[...], wf_ref[...], preferred_element_type=jnp.float32)
    acc += lax.dot_general(
        embt_ref[...],
        w2_ref[...],
        dimension_numbers=(((0,), (0,)), ((), ())),
        preferred_element_type=jnp.float32,
    )
    out_ref[...] = acc + b_ref[...]


def _tc_matmul(feat_pad, embt, wf, w2, b):
    return pl.pallas_call(
        _mm_body,
        grid=(NPAD // BN,),
        in_specs=[
            pl.BlockSpec((BN, IN_FEATS), lambda i: (i, 0)),
            pl.BlockSpec((EMB_COLS, BN), lambda i: (0, i)),
            pl.BlockSpec((IN_FEATS, OUT), lambda i: (0, 0)),
            pl.BlockSpec((EMB_COLS, OUT), lambda i: (0, 0)),
            pl.BlockSpec((1, OUT), lambda i: (0, 0)),
        ],
        out_specs=pl.BlockSpec((BN, OUT), lambda i: (i, 0)),
        out_shape=jax.ShapeDtypeStruct((NPAD, OUT), jnp.float32),
    )(feat_pad, embt, wf, w2, b)


def kernel(g, features, tables, W, b):
    idx = features[:, :NCAT].astype(jnp.int32)
    idx_t = jnp.pad(idx.T, ((0, 0), (0, NPAD - N)))
    feat_pad = jnp.pad(features, ((0, NPAD - N), (0, 0)))
    tb = tables.astype(jnp.bfloat16)
    tables8 = jax.lax.bitcast_convert_type(
        tb.reshape(NCAT, GPC, 128, 2), jnp.int32
    )
    wf = jnp.concatenate([jnp.zeros((NCAT, OUT), jnp.float32), W[: IN_FEATS - NCAT]])
    w2 = W[IN_FEATS - NCAT:]

    embt = _sc_gather(tables8, idx_t)
    out = _tc_matmul(feat_pad, embt, wf, w2, b.reshape(1, OUT))
    return out[:N]


# speedup vs baseline: 1.0033x; 1.0033x over previous
import functools

import jax
import jax.numpy as jnp
from jax import lax
from jax.experimental import pallas as pl
from jax.experimental.pallas import tpu as pltpu
from jax.experimental.pallas import tpu_sc as plsc

N = 100000
IN_FEATS = 128
NCAT = 26
V = 100000
D = 16
OUT = 128
EMB_COLS = NCAT * D
GPC = V // 16

NW = 32
BPW = 3200
NPAD = NW * BPW
R = 128
NCHUNK = BPW // R
BN = 2048


def _sc_gather(tables8, idx_t):
    mesh = plsc.VectorSubcoreMesh(core_axis_name="c", subcore_axis_name="s")
    nc = 2

    @functools.partial(
        pl.kernel,
        mesh=mesh,
        out_type=jax.ShapeDtypeStruct((EMB_COLS, NPAD), jnp.float32),
        scratch_types=[
            pltpu.VMEM((2, R), jnp.int32),
            pltpu.VMEM((2, R), jnp.int32),
            pltpu.VMEM((2, R, 128), jnp.int32),
            pltpu.VMEM((D, R), jnp.float32),
            pltpu.VMEM_SHARED((GPC, 128), jnp.int32),
            pltpu.SemaphoreType.DMA,
        ],
        compiler_params=pltpu.CompilerParams(needs_layout_passes=False),
    )
    def k(t8_hbm, idx_hbm, emb_hbm, idx_v, gidx_v, sup_v, pk_v, tab_sh, gsem):
        cid = lax.axis_index("c")
        sid = lax.axis_index("s")
        wid = sid * nc + cid
        base = pl.multiple_of(wid * BPW, 128)
        iota16 = lax.iota(jnp.int32, 16)

        def load_prep_fire(c, r):
            h = r & 1
            off = pl.multiple_of(base + r * R, 128)
            pltpu.sync_copy(idx_hbm.at[c].at[pl.ds(off, R)], idx_v.at[h])

            def prep8(i8, _):
                v = idx_v[h, pl.ds(i8 * 16, 16)]
                gidx_v[h, pl.ds(i8 * 16, 16)] = v >> 4
                return _

            lax.fori_loop(0, R // 16, prep8, 0)
            pltpu.async_copy(tab_sh.at[gidx_v.at[h]], sup_v.at[h], gsem)

        def drain_gather():
            pltpu.make_async_copy(
                t8_hbm.at[0].at[pl.ds(0, R), :], sup_v.at[0], gsem
            ).wait()

        def compact_wb(c, r):
            h = r & 1

            def body(i8, _):
                rows = i8 * 16 + iota16
                kv = (idx_v[h, pl.ds(i8 * 16, 16)] & 15) * 8
                for j in range(D // 2):
                    vals = plsc.load_gather(sup_v.at[h], [rows, kv + j])
                    lo = plsc.bitcast(vals << 16, jnp.float32)
                    hi = plsc.bitcast(vals & jnp.int32(-65536), jnp.float32)
                    plsc.store_scatter(
                        pk_v,
                        [jnp.broadcast_to(jnp.int32(2 * j), (16,)), rows],
                        lo,
                    )
                    plsc.store_scatter(
                        pk_v,
                        [jnp.broadcast_to(jnp.int32(2 * j + 1), (16,)), rows],
                        hi,
                    )
                return _

            lax.fori_loop(0, R // 16, body, 0)
            off = pl.multiple_of(base + r * R, 128)
            pltpu.sync_copy(
                pk_v,
                emb_hbm.at[pl.ds(pl.multiple_of(c * D, 8), D), pl.ds(off, R)],
            )

        def col(c, carry):
            @pl.when(sid == 0)
            def _():
                pltpu.sync_copy(t8_hbm.at[c], tab_sh)

            plsc.subcore_barrier()

            load_prep_fire(c, 0)

            def step(r, cc):
                @pl.when(r + 1 < NCHUNK)
                def _():
                    load_prep_fire(c, r + 1)

                drain_gather()
                compact_wb(c, r)
                return cc

            lax.fori_loop(0, NCHUNK, step, 0)
            plsc.subcore_barrier()
            return carry

        lax.fori_loop(0, NCAT, col, 0)

    return k(tables8, idx_t)


def _mm_body(feat_ref, embt_ref, wf_ref, w2_ref, b_ref, out_ref):
    acc = jnp.dot(feat_ref[...], wf_ref[...], preferred_element_type=jnp.float32)
    acc += lax.dot_general(
        embt_ref[...],
        w2_ref[...],
        dimension_numbers=(((0,), (0,)), ((), ())),
        preferred_element_type=jnp.float32,
    )
    out_ref[...] = acc + b_ref[...]


def _tc_matmul(feat_pad, embt, wf, w2, b):
    return pl.pallas_call(
        _mm_body,
        grid=(NPAD // BN,),
        in_specs=[
            pl.BlockSpec((BN, IN_FEATS), lambda i: (i, 0)),
            pl.BlockSpec((EMB_COLS, BN), lambda i: (0, i)),
            pl.BlockSpec((IN_FEATS, OUT), lambda i: (0, 0)),
            pl.BlockSpec((EMB_COLS, OUT), lambda i: (0, 0)),
            pl.BlockSpec((1, OUT), lambda i: (0, 0)),
        ],
        out_specs=pl.BlockSpec((BN, OUT), lambda i: (i, 0)),
        out_shape=jax.ShapeDtypeStruct((NPAD, OUT), jnp.float32),
    )(feat_pad, embt, wf, w2, b)


def kernel(g, features, tables, W, b):
    idx = features[:, :NCAT].astype(jnp.int32)
    idx_t = jnp.pad(idx.T, ((0, 0), (0, NPAD - N)))
    feat_pad = jnp.pad(features, ((0, NPAD - N), (0, 0)))
    tb = tables.astype(jnp.bfloat16)
    tables8 = jax.lax.bitcast_convert_type(
        tb.reshape(NCAT, GPC, 128, 2), jnp.int32
    )
    wf = jnp.concatenate([jnp.zeros((NCAT, OUT), jnp.float32), W[: IN_FEATS - NCAT]])
    w2 = W[IN_FEATS - NCAT:]

    embt = _sc_gather(tables8, idx_t)
    out = _tc_matmul(feat_pad, embt, wf, w2, b.reshape(1, OUT))
    return out[:N]


# speedup vs baseline: 1.1039x; 1.1002x over previous
import functools

import jax
import jax.numpy as jnp
from jax import lax
from jax.experimental import pallas as pl
from jax.experimental.pallas import tpu as pltpu
from jax.experimental.pallas import tpu_sc as plsc

N = 100000
IN_FEATS = 128
NCAT = 26
V = 100000
D = 16
OUT = 128
EMB_COLS = NCAT * D
GPC = V // 8

NW = 32
BPW = 3200
NPAD = NW * BPW
R = 128
NCHUNK = BPW // R
BN = 2048


def _sc_gather(tables8, idx_t):
    mesh = plsc.VectorSubcoreMesh(core_axis_name="c", subcore_axis_name="s")
    nc = 2

    @functools.partial(
        pl.kernel,
        mesh=mesh,
        out_type=jax.ShapeDtypeStruct((EMB_COLS, NPAD), jnp.float32),
        scratch_types=[
            pltpu.VMEM((R,), jnp.int32),
            pltpu.VMEM((R,), jnp.int32),
            pltpu.VMEM((R, 128), jnp.float32),
            pltpu.VMEM((D, R), jnp.float32),
            pltpu.VMEM_SHARED((GPC, 128), jnp.float32),
            pltpu.SemaphoreType.DMA,
        ],
        compiler_params=pltpu.CompilerParams(needs_layout_passes=False),
    )
    def k(t8_hbm, idx_hbm, emb_hbm, idx_v, gidx_v, super_v, packt_v, tab_sh, sem):
        cid = lax.axis_index("c")
        sid = lax.axis_index("s")
        wid = sid * nc + cid
        base = pl.multiple_of(wid * BPW, 128)
        iota16 = lax.iota(jnp.int32, 16)

        def col(c, carry):
            @pl.when(sid == 0)
            def _():
                pltpu.sync_copy(t8_hbm.at[c], tab_sh)

            plsc.subcore_barrier()

            def chunk(r, cc):
                off = pl.multiple_of(base + r * R, 128)
                pltpu.sync_copy(idx_hbm.at[c].at[pl.ds(off, R)], idx_v)

                def prep8(i8, _):
                    v = idx_v[pl.ds(i8 * 16, 16)]
                    gidx_v[pl.ds(i8 * 16, 16)] = v >> 3
                    return _

                lax.fori_loop(0, R // 16, prep8, 0)
                pltpu.async_copy(tab_sh.at[gidx_v], super_v, sem).wait()

                def body(i8, _):
                    rows = i8 * 16 + iota16
                    kv = (idx_v[pl.ds(i8 * 16, 16)] & 7) * 16
                    for d in range(D):
                        vals = plsc.load_gather(super_v, [rows, kv + d])
                        plsc.store_scatter(
                            packt_v,
                            [jnp.broadcast_to(jnp.int32(d), (16,)), rows],
                            vals,
                        )
                    return _

                lax.fori_loop(0, R // 16, body, 0)
                pltpu.sync_copy(
                    packt_v,
                    emb_hbm.at[
                        pl.ds(pl.multiple_of(c * D, 8), D), pl.ds(off, R)
                    ],
                )
                return cc

            lax.fori_loop(0, NCHUNK, chunk, 0)
            plsc.subcore_barrier()
            return carry

        lax.fori_loop(0, NCAT, col, 0)

    return k(tables8, idx_t)


def _mm_body(feat_ref, embt_ref, wf_ref, w2_ref, b_ref, out_ref):
    acc = jnp.dot(feat_ref[...], wf_ref[...], preferred_element_type=jnp.float32)
    acc += lax.dot_general(
        embt_ref[...],
        w2_ref[...],
        dimension_numbers=(((0,), (0,)), ((), ())),
        preferred_element_type=jnp.float32,
    )
    out_ref[...] = acc + b_ref[...]


def _tc_matmul(feat_pad, embt, wf, w2, b):
    return pl.pallas_call(
        _mm_body,
        grid=(NPAD // BN,),
        in_specs=[
            pl.BlockSpec((BN, IN_FEATS), lambda i: (i, 0)),
            pl.BlockSpec((EMB_COLS, BN), lambda i: (0, i)),
            pl.BlockSpec((IN_FEATS, OUT), lambda i: (0, 0)),
            pl.BlockSpec((EMB_COLS, OUT), lambda i: (0, 0)),
            pl.BlockSpec((1, OUT), lambda i: (0, 0)),
        ],
        out_specs=pl.BlockSpec((BN, OUT), lambda i: (i, 0)),
        out_shape=jax.ShapeDtypeStruct((NPAD, OUT), jnp.float32),
    )(feat_pad, embt, wf, w2, b)


def kernel(g, features, tables, W, b):
    idx = features[:, :NCAT].astype(jnp.int32)
    idx_t = jnp.pad(idx.T, ((0, 0), (0, NPAD - N)))
    feat_pad = jnp.pad(features, ((0, NPAD - N), (0, 0)))
    tables8 = tables.reshape(NCAT, GPC, 128)
    wf = jnp.concatenate([jnp.zeros((NCAT, OUT), jnp.float32), W[: IN_FEATS - NCAT]])
    w2 = W[IN_FEATS - NCAT:]

    embt = _sc_gather(tables8, idx_t)
    out = _tc_matmul(feat_pad, embt, wf, w2, b.reshape(1, OUT))
    return out[:N]
